# Initial kernel scaffold; baseline (speedup 1.0000x reference)
import jax
import jax.numpy as jnp
from jax.experimental import pallas as pl


def kernel(x, vertices, hyperedges, edge_features, eW0, eb0, eW1, eb1, eW2, eb2, eg, ebt, nW0, nb0, nW1, nb1, nW2, nb2, ng, nbt):
    raise NotImplementedError("write your pallas kernel here")


# speedup vs baseline: 2.4245x; 2.4245x over previous
import functools

import jax
import jax.numpy as jnp
from jax import lax
from jax.experimental import pallas as pl
from jax.experimental.pallas import tpu as pltpu
from jax.experimental.pallas import tpu_sc as plsc

N = 10000
I = 320000
E = 160000
D = 128
NC = 2
NS = 16
W = 2048
NWIN = 80
HALF = N // NC
NACC = 5120
CH = 128
PAD = 256
L = 16


def _zero_vmem_rows(buf, nrows):
    z = jnp.zeros((L,), jnp.float32)

    def row(r, _):
        for q in range(D // L):
            buf[r, pl.ds(q * L, L)] = z
        return 0

    lax.fori_loop(0, nrows, row, 0)


def _extract(wsv, j):
    return wsv[pl.ds(j, L)][0]


def _mask_chunk(dstv, i0, lo, hi, garbage, base=None, limit=None):
    i16 = lax.iota(jnp.int32, 16)
    for q in range(CH // L):
        dv = dstv[pl.ds(q * L, L)]
        pos = i0 + q * L + i16
        ok = (pos >= lo) & (pos < hi)
        if base is not None:
            dv = dv - base
            ok = ok & (dv >= 0) & (dv < limit)
        dstv[pl.ds(q * L, L)] = jnp.where(ok, dv, garbage)


def _seg_chunk_body(src_hbm, idx_hbm, dst_hbm, vidx, dstv, rows, acc, sem,
                    i0, lo, hi, garbage, base=None, limit=None):
    i0 = pl.multiple_of(i0, 8)
    pltpu.sync_copy(idx_hbm.at[pl.ds(i0, CH)], vidx)
    pltpu.sync_copy(dst_hbm.at[pl.ds(i0, CH)], dstv)
    _mask_chunk(dstv, i0, lo, hi, garbage, base, limit)
    pltpu.async_copy(src_hbm.at[vidx], rows, sem).wait()
    pltpu.sync_copy(rows, acc.at[dstv], add=True)


def _edge_segsum_kernel():
    mesh = plsc.VectorSubcoreMesh(core_axis_name="c", subcore_axis_name="s")
    rows_per_tile = W // NS

    @functools.partial(
        pl.kernel,
        out_type=jax.ShapeDtypeStruct((NWIN * W, D), jnp.float32),
        mesh=mesh,
        scratch_types=[
            pltpu.VMEM((CH,), jnp.int32),
            pltpu.VMEM((CH,), jnp.int32),
            pltpu.VMEM((CH, D), jnp.float32),
            pltpu.VMEM((rows_per_tile, D), jnp.float32),
            pltpu.VMEM((96,), jnp.int32),
            pltpu.VMEM_SHARED((W + 8, D), jnp.float32),
            pltpu.SemaphoreType.DMA,
        ],
    )
    def k(x_hbm, vert_hbm, hl_hbm, ws_hbm, xe_hbm,
          vidx, dstv, rows, zbuf, wsv, acc, sem):
        c = lax.axis_index("c")
        s = lax.axis_index("s")
        pltpu.sync_copy(ws_hbm, wsv)
        _zero_vmem_rows(zbuf, rows_per_tile)
        my_rows = s * rows_per_tile

        for wi in range(NWIN // NC):
            w = wi * NC + c
            lo_all = _extract(wsv, w)
            hi_all = _extract(wsv, w + 1)
            pltpu.sync_copy(zbuf, acc.at[pl.ds(my_rows, rows_per_tile)])
            plsc.subcore_barrier()
            cnt = hi_all - lo_all
            lo = lo_all + ((s * cnt) >> 4)
            hi = lo_all + (((s + 1) * cnt) >> 4)
            lo8 = lo & (-8)
            nch = (hi - lo8 + CH - 1) >> 7

            def chunk(kk, _):
                i0 = lo8 + kk * CH
                _seg_chunk_body(x_hbm, vert_hbm, hl_hbm, vidx, dstv, rows,
                                acc, sem, i0, lo, hi, W)
                return 0

            lax.fori_loop(0, nch, chunk, 0)
            plsc.subcore_barrier()
            pltpu.sync_copy(acc.at[pl.ds(my_rows, rows_per_tile)],
                            xe_hbm.at[pl.ds(w * W + my_rows, rows_per_tile)])
            plsc.subcore_barrier()

    return k


def _node_segsum_kernel():
    mesh = plsc.VectorSubcoreMesh(core_axis_name="c", subcore_axis_name="s")
    per_tile = I // NS
    z_rows = NACC // NS
    nch = (per_tile + CH - 1) // CH

    @functools.partial(
        pl.kernel,
        out_type=jax.ShapeDtypeStruct((NC, NACC, D), jnp.float32),
        mesh=mesh,
        scratch_types=[
            pltpu.VMEM((CH,), jnp.int32),
            pltpu.VMEM((CH,), jnp.int32),
            pltpu.VMEM((CH, D), jnp.float32),
            pltpu.VMEM((z_rows, D), jnp.float32),
            pltpu.VMEM_SHARED((NACC, D), jnp.float32),
            pltpu.SemaphoreType.DMA,
        ],
    )
    def k(ue_hbm, he_hbm, vert_hbm, xv_hbm, vidx, dstv, rows, zbuf, acc, sem):
        c = lax.axis_index("c")
        s = lax.axis_index("s")
        _zero_vmem_rows(zbuf, z_rows)
        pltpu.sync_copy(zbuf, acc.at[pl.ds(s * z_rows, z_rows)])
        plsc.subcore_barrier()
        lo = s * per_tile
        hi = lo + per_tile
        base = c * HALF

        def chunk(kk, _):
            i0 = lo + kk * CH
            _seg_chunk_body(ue_hbm, he_hbm, vert_hbm, vidx, dstv, rows,
                            acc, sem, i0, lo, hi, HALF, base, HALF)
            return 0

        lax.fori_loop(0, nch, chunk, 0)
        plsc.subcore_barrier()
        pltpu.sync_copy(acc.at[pl.ds(s * z_rows, z_rows)],
                        xv_hbm.at[c, pl.ds(s * z_rows, z_rows)])

    return k


BE = 640
BN = 1000


def _edge_mlp_body(xe, ef, w0a, w0b, b0, w1, b1, w2, b2, g, bt, upd, eo):
    h = jnp.dot(xe[...], w0a[...], preferred_element_type=jnp.float32)
    h = h + jnp.dot(ef[...], w0b[...], preferred_element_type=jnp.float32)
    h = jnp.maximum(h + b0[...], 0.0)
    h = jnp.maximum(jnp.dot(h, w1[...], preferred_element_type=jnp.float32) + b1[...], 0.0)
    h = jnp.dot(h, w2[...], preferred_element_type=jnp.float32) + b2[...]
    m = jnp.mean(h, axis=-1, keepdims=True)
    v = jnp.mean((h - m) ** 2, axis=-1, keepdims=True)
    u = (h - m) * lax.rsqrt(v + 1e-5) * g[...] + bt[...]
    upd[...] = u
    eo[...] = u + ef[...]


def _node_mlp_body(xv, x, w0a, w0b, b0, w1, b1, w2, b2, g, bt, xo):
    h = jnp.dot(xv[0], w0a[...], preferred_element_type=jnp.float32)
    h = h + jnp.dot(x[...], w0b[...], preferred_element_type=jnp.float32)
    h = jnp.maximum(h + b0[...], 0.0)
    h = jnp.maximum(jnp.dot(h, w1[...], preferred_element_type=jnp.float32) + b1[...], 0.0)
    h = jnp.dot(h, w2[...], preferred_element_type=jnp.float32) + b2[...]
    m = jnp.mean(h, axis=-1, keepdims=True)
    v = jnp.mean((h - m) ** 2, axis=-1, keepdims=True)
    u = (h - m) * lax.rsqrt(v + 1e-5) * g[...] + bt[...]
    xo[...] = u + x[...]


def _row_spec(rows):
    return pl.BlockSpec((rows, D), lambda i: (i, 0))


def _full_spec(shape):
    return pl.BlockSpec(shape, lambda i: tuple(0 for _ in shape))


def _edge_mlp(xe, ef, w0a, w0b, b0, w1, b1, w2, b2, g, bt):
    specs = ([_row_spec(BE), _row_spec(BE)]
             + [_full_spec(w.shape) for w in (w0a, w0b, b0, w1, b1, w2, b2, g, bt)])
    return pl.pallas_call(
        _edge_mlp_body,
        grid=(E // BE,),
        in_specs=specs,
        out_specs=(_row_spec(BE), _row_spec(BE)),
        out_shape=(jax.ShapeDtypeStruct((E, D), jnp.float32),
                   jax.ShapeDtypeStruct((E, D), jnp.float32)),
        compiler_params=pltpu.CompilerParams(dimension_semantics=("parallel",)),
    )(xe, ef, w0a, w0b, b0, w1, b1, w2, b2, g, bt)


def _node_mlp(xv, x, w0a, w0b, b0, w1, b1, w2, b2, g, bt):
    nb = HALF // BN
    xv_spec = pl.BlockSpec((1, BN, D), lambda i: (i // nb, i % nb, 0))
    specs = ([xv_spec, _row_spec(BN)]
             + [_full_spec(w.shape) for w in (w0a, w0b, b0, w1, b1, w2, b2, g, bt)])
    return pl.pallas_call(
        _node_mlp_body,
        grid=(N // BN,),
        in_specs=specs,
        out_specs=_row_spec(BN),
        out_shape=jax.ShapeDtypeStruct((N, D), jnp.float32),
        compiler_params=pltpu.CompilerParams(dimension_semantics=("parallel",)),
    )(xv, x, w0a, w0b, b0, w1, b1, w2, b2, g, bt)


def kernel(x, vertices, hyperedges, edge_features,
           eW0, eb0, eW1, eb1, eW2, eb2, eg, ebt,
           nW0, nb0, nW1, nb1, nW2, nb2, ng, nbt):
    S = eW0.shape[0]
    ipad = jnp.zeros((PAD,), jnp.int32)
    vert_p = jnp.concatenate([vertices, ipad])
    he_p = jnp.concatenate([hyperedges, ipad])
    hl_p = jnp.concatenate([hyperedges % W, ipad])
    bnd = jnp.searchsorted(
        hyperedges, jnp.arange(NWIN + 1, dtype=jnp.int32) * W, side='left'
    ).astype(jnp.int32)
    ws = jnp.full((96,), I, jnp.int32).at[:NWIN + 1].set(bnd)

    edge_seg = _edge_segsum_kernel()
    node_seg = _node_segsum_kernel()

    for s in range(S):
        xe = edge_seg(x, vert_p, hl_p, ws)
        upd_e, e_out = _edge_mlp(
            xe, edge_features,
            eW0[s][:D], eW0[s][D:], eb0[s][None, :],
            eW1[s], eb1[s][None, :], eW2[s], eb2[s][None, :],
            eg[s][None, :], ebt[s][None, :])
        xv = node_seg(upd_e, he_p, vert_p)
        x = _node_mlp(
            xv, x,
            nW0[s][:D], nW0[s][D:], nb0[s][None, :],
            nW1[s], nb1[s][None, :], nW2[s], nb2[s][None, :],
            ng[s][None, :], nbt[s][None, :])
        edge_features = e_out

    return (x, edge_features)


# speedup vs baseline: 2.9057x; 1.1985x over previous
import functools

import jax
import jax.numpy as jnp
from jax import lax
from jax.experimental import pallas as pl
from jax.experimental.pallas import tpu as pltpu
from jax.experimental.pallas import tpu_sc as plsc

N = 10000
I = 320000
E = 160000
D = 128
NC = 2
NS = 16
W = 2048
NWIN = 80
HALF = N // NC
NACC = 5120
CH = 128
PAD = 512
L = 16


def _zero_vmem_rows(buf, nrows):
    z = jnp.zeros((L,), jnp.float32)

    def row(r, _):
        for q in range(D // L):
            buf[r, pl.ds(q * L, L)] = z
        return 0

    lax.fori_loop(0, nrows, row, 0)


def _extract(wsv, j):
    return wsv[pl.ds(j, L)][0]


def _mask_chunk(dstv, i0, lo, hi, garbage, base=None, limit=None):
    i16 = lax.iota(jnp.int32, 16)
    for q in range(CH // L):
        dv = dstv[pl.ds(q * L, L)]
        pos = i0 + q * L + i16
        ok = (pos >= lo) & (pos < hi)
        if base is not None:
            dv = dv - base
            ok = ok & (dv >= 0) & (dv < limit)
        dstv[pl.ds(q * L, L)] = jnp.where(ok, dv, garbage)


def _seg_chunk_body(src_hbm, idx_hbm, dst_hbm, vidx, dstv, rows, acc, sem,
                    i0, lo, hi, garbage, base=None, limit=None):
    i0 = pl.multiple_of(i0, 8)
    pltpu.sync_copy(idx_hbm.at[pl.ds(i0, CH)], vidx)
    pltpu.sync_copy(dst_hbm.at[pl.ds(i0, CH)], dstv)
    _mask_chunk(dstv, i0, lo, hi, garbage, base, limit)
    pltpu.async_copy(src_hbm.at[vidx], rows, sem).wait()
    pltpu.sync_copy(rows, acc.at[dstv], add=True)


def _edge_segsum_kernel():
    mesh = plsc.VectorSubcoreMesh(core_axis_name="c", subcore_axis_name="s")
    rows_per_tile = W // NS

    @functools.partial(
        pl.kernel,
        out_type=jax.ShapeDtypeStruct((NWIN * W, D), jnp.float32),
        mesh=mesh,
        scratch_types=[
            pltpu.VMEM((CH,), jnp.int32),
            pltpu.VMEM((CH,), jnp.int32),
            pltpu.VMEM((CH, D), jnp.float32),
            pltpu.VMEM((rows_per_tile, D), jnp.float32),
            pltpu.VMEM((96,), jnp.int32),
            pltpu.VMEM_SHARED((W + 8, D), jnp.float32),
            pltpu.SemaphoreType.DMA,
        ],
    )
    def k(x_hbm, vert_hbm, hl_hbm, ws_hbm, xe_hbm,
          vidx, dstv, rows, zbuf, wsv, acc, sem):
        c = lax.axis_index("c")
        s = lax.axis_index("s")
        pltpu.sync_copy(ws_hbm, wsv)
        _zero_vmem_rows(zbuf, rows_per_tile)
        my_rows = s * rows_per_tile

        for wi in range(NWIN // NC):
            w = wi * NC + c
            lo_all = _extract(wsv, w)
            hi_all = _extract(wsv, w + 1)
            pltpu.sync_copy(zbuf, acc.at[pl.ds(my_rows, rows_per_tile)])
            plsc.subcore_barrier()
            cnt = hi_all - lo_all
            lo = lo_all + ((s * cnt) >> 4)
            hi = lo_all + (((s + 1) * cnt) >> 4)
            lo8 = lo & (-8)
            nch = (hi - lo8 + CH - 1) >> 7

            def chunk(kk, _):
                i0 = lo8 + kk * CH
                _seg_chunk_body(x_hbm, vert_hbm, hl_hbm, vidx, dstv, rows,
                                acc, sem, i0, lo, hi, W)
                return 0

            lax.fori_loop(0, nch, chunk, 0)
            plsc.subcore_barrier()
            pltpu.sync_copy(acc.at[pl.ds(my_rows, rows_per_tile)],
                            xe_hbm.at[pl.ds(w * W + my_rows, rows_per_tile)])
            plsc.subcore_barrier()

    return k


def _node_segsum_kernel():
    mesh = plsc.VectorSubcoreMesh(core_axis_name="c", subcore_axis_name="s")
    per_tile = I // NS
    z_rows = NACC // NS
    GD = 4
    NCH = 160
    HCH = NCH // 2
    IDXN = HCH * CH

    @functools.partial(
        pl.kernel,
        out_type=jax.ShapeDtypeStruct((NC, NACC, D), jnp.float32),
        mesh=mesh,
        scratch_types=[
            pltpu.VMEM((IDXN,), jnp.int32),
            pltpu.VMEM((IDXN,), jnp.int32),
            pltpu.VMEM((GD, CH), jnp.int32),
            pltpu.VMEM((CH, D), jnp.float32),
            pltpu.VMEM((CH, D), jnp.float32),
            pltpu.VMEM((CH, D), jnp.float32),
            pltpu.VMEM((CH, D), jnp.float32),
            pltpu.VMEM_SHARED((NACC, D), jnp.float32),
            pltpu.SemaphoreType.DMA,
            pltpu.SemaphoreType.DMA,
        ],
    )
    def k(ue_hbm, he_hbm, vert_hbm, xv_hbm, gidx, didx, dstv2, r0, r1, r2, r3,
          acc, gsem, ssem):
        rows_b = (r0, r1, r2, r3)
        c = lax.axis_index("c")
        s = lax.axis_index("s")
        lo = s * per_tile
        hi = lo + per_tile
        base = c * HALF

        for b in range(GD):
            _zero_vmem_rows(rows_b[b], CH if b < 2 else z_rows - 2 * CH)
        pltpu.sync_copy(r0, acc.at[pl.ds(s * z_rows, CH)])
        pltpu.sync_copy(r1, acc.at[pl.ds(s * z_rows + CH, CH)])
        pltpu.sync_copy(r2.at[pl.ds(0, z_rows - 2 * CH)],
                        acc.at[pl.ds(s * z_rows + 2 * CH, z_rows - 2 * CH)])
        plsc.subcore_barrier()
        pltpu.sync_copy(he_hbm.at[pl.ds(lo, IDXN)], gidx)
        pltpu.sync_copy(vert_hbm.at[pl.ds(lo, IDXN)], didx)

        def gather_desc(kl, b):
            off = pl.multiple_of(kl * CH, CH)
            return pltpu.make_async_copy(
                ue_hbm.at[gidx.at[pl.ds(off, CH)]], rows_b[b], gsem)

        def scatter_desc(b):
            return pltpu.make_async_copy(rows_b[b], acc.at[dstv2.at[b]], ssem)

        def mask_into(kk, kl, b):
            i16 = lax.iota(jnp.int32, 16)
            koff = pl.multiple_of(kl * CH, CH)
            for q in range(CH // L):
                dv = didx[pl.ds(koff + q * L, L)] - base
                pos = lo + kk * CH + q * L + i16
                ok = (pos < hi) & (dv >= 0) & (dv < HALF)
                dstv2[b, pl.ds(q * L, L)] = jnp.where(ok, dv, HALF)

        half_groups = HCH // GD

        def group(gi, _):
            @pl.when(gi == half_groups)
            def _():
                pltpu.sync_copy(he_hbm.at[pl.ds(lo + IDXN, IDXN)], gidx)
                pltpu.sync_copy(vert_hbm.at[pl.ds(lo + IDXN, IDXN)], didx)
            k0 = gi * GD
            kl0 = jnp.where(gi >= half_groups, k0 - HCH, k0)
            for b in range(GD):
                @pl.when(gi > 0)
                def _():
                    scatter_desc(b).wait()
                mask_into(k0 + b, kl0 + b, b)
                gather_desc(kl0 + b, b).start()
            for b in range(GD):
                gather_desc(kl0 + b, b).wait()
                scatter_desc(b).start(add=True)
            return 0

        lax.fori_loop(0, NCH // GD, group, 0)
        for b in range(GD):
            scatter_desc(b).wait()
        plsc.subcore_barrier()
        pltpu.sync_copy(acc.at[pl.ds(s * z_rows, z_rows)],
                        xv_hbm.at[c, pl.ds(s * z_rows, z_rows)])

    return k


BE = 640
BN = 1000


def _edge_mlp_body(xe, ef, w0a, w0b, b0, w1, b1, w2, b2, g, bt, upd, eo):
    h = jnp.dot(xe[...], w0a[...], preferred_element_type=jnp.float32)
    h = h + jnp.dot(ef[...], w0b[...], preferred_element_type=jnp.float32)
    h = jnp.maximum(h + b0[...], 0.0)
    h = jnp.maximum(jnp.dot(h, w1[...], preferred_element_type=jnp.float32) + b1[...], 0.0)
    h = jnp.dot(h, w2[...], preferred_element_type=jnp.float32) + b2[...]
    m = jnp.mean(h, axis=-1, keepdims=True)
    v = jnp.mean((h - m) ** 2, axis=-1, keepdims=True)
    u = (h - m) * lax.rsqrt(v + 1e-5) * g[...] + bt[...]
    upd[...] = u
    eo[...] = u + ef[...]


def _node_mlp_body(xv, x, w0a, w0b, b0, w1, b1, w2, b2, g, bt, xo):
    h = jnp.dot(xv[0], w0a[...], preferred_element_type=jnp.float32)
    h = h + jnp.dot(x[...], w0b[...], preferred_element_type=jnp.float32)
    h = jnp.maximum(h + b0[...], 0.0)
    h = jnp.maximum(jnp.dot(h, w1[...], preferred_element_type=jnp.float32) + b1[...], 0.0)
    h = jnp.dot(h, w2[...], preferred_element_type=jnp.float32) + b2[...]
    m = jnp.mean(h, axis=-1, keepdims=True)
    v = jnp.mean((h - m) ** 2, axis=-1, keepdims=True)
    u = (h - m) * lax.rsqrt(v + 1e-5) * g[...] + bt[...]
    xo[...] = u + x[...]


def _row_spec(rows):
    return pl.BlockSpec((rows, D), lambda i: (i, 0))


def _full_spec(shape):
    return pl.BlockSpec(shape, lambda i: tuple(0 for _ in shape))


def _edge_mlp(xe, ef, w0a, w0b, b0, w1, b1, w2, b2, g, bt):
    specs = ([_row_spec(BE), _row_spec(BE)]
             + [_full_spec(w.shape) for w in (w0a, w0b, b0, w1, b1, w2, b2, g, bt)])
    return pl.pallas_call(
        _edge_mlp_body,
        grid=(E // BE,),
        in_specs=specs,
        out_specs=(_row_spec(BE), _row_spec(BE)),
        out_shape=(jax.ShapeDtypeStruct((E, D), jnp.float32),
                   jax.ShapeDtypeStruct((E, D), jnp.float32)),
        compiler_params=pltpu.CompilerParams(dimension_semantics=("parallel",)),
    )(xe, ef, w0a, w0b, b0, w1, b1, w2, b2, g, bt)


def _node_mlp(xv, x, w0a, w0b, b0, w1, b1, w2, b2, g, bt):
    nb = HALF // BN
    xv_spec = pl.BlockSpec((1, BN, D), lambda i: (i // nb, i % nb, 0))
    specs = ([xv_spec, _row_spec(BN)]
             + [_full_spec(w.shape) for w in (w0a, w0b, b0, w1, b1, w2, b2, g, bt)])
    return pl.pallas_call(
        _node_mlp_body,
        grid=(N // BN,),
        in_specs=specs,
        out_specs=_row_spec(BN),
        out_shape=jax.ShapeDtypeStruct((N, D), jnp.float32),
        compiler_params=pltpu.CompilerParams(dimension_semantics=("parallel",)),
    )(xv, x, w0a, w0b, b0, w1, b1, w2, b2, g, bt)


def kernel(x, vertices, hyperedges, edge_features,
           eW0, eb0, eW1, eb1, eW2, eb2, eg, ebt,
           nW0, nb0, nW1, nb1, nW2, nb2, ng, nbt):
    S = eW0.shape[0]
    ipad = jnp.zeros((PAD,), jnp.int32)
    vert_p = jnp.concatenate([vertices, ipad])
    he_p = jnp.concatenate([hyperedges, ipad])
    hl_p = jnp.concatenate([hyperedges % W, ipad])
    bnd = jnp.searchsorted(
        hyperedges, jnp.arange(NWIN + 1, dtype=jnp.int32) * W, side='left'
    ).astype(jnp.int32)
    ws = jnp.full((96,), I, jnp.int32).at[:NWIN + 1].set(bnd)

    edge_seg = _edge_segsum_kernel()
    node_seg = _node_segsum_kernel()

    for s in range(S):
        xe = edge_seg(x, vert_p, hl_p, ws)
        upd_e, e_out = _edge_mlp(
            xe, edge_features,
            eW0[s][:D], eW0[s][D:], eb0[s][None, :],
            eW1[s], eb1[s][None, :], eW2[s], eb2[s][None, :],
            eg[s][None, :], ebt[s][None, :])
        xv = node_seg(upd_e, he_p, vert_p)
        x = _node_mlp(
            xv, x,
            nW0[s][:D], nW0[s][D:], nb0[s][None, :],
            nW1[s], nb1[s][None, :], nW2[s], nb2[s][None, :],
            ng[s][None, :], nbt[s][None, :])
        edge_features = e_out

    return (x, edge_features)


# speedup vs baseline: 3.3000x; 1.1357x over previous
import functools

import jax
import jax.numpy as jnp
from jax import lax
from jax.experimental import pallas as pl
from jax.experimental.pallas import tpu as pltpu
from jax.experimental.pallas import tpu_sc as plsc

N = 10000
I = 320000
E = 160000
D = 128
NC = 2
NS = 16
W = 2048
NWIN = 80
HALF = N // NC
NACC = 5120
CH = 128
PAD = 2048
L = 16


def _zero_vmem_rows(buf, nrows):
    z = jnp.zeros((L,), jnp.float32)

    def row(r, _):
        for q in range(D // L):
            buf[r, pl.ds(q * L, L)] = z
        return 0

    lax.fori_loop(0, nrows, row, 0)


def _extract(wsv, j):
    return wsv[pl.ds(j, L)][0]


def _mask_chunk(dstv, i0, lo, hi, garbage, base=None, limit=None):
    i16 = lax.iota(jnp.int32, 16)
    for q in range(CH // L):
        dv = dstv[pl.ds(q * L, L)]
        pos = i0 + q * L + i16
        ok = (pos >= lo) & (pos < hi)
        if base is not None:
            dv = dv - base
            ok = ok & (dv >= 0) & (dv < limit)
        dstv[pl.ds(q * L, L)] = jnp.where(ok, dv, garbage)


def _seg_chunk_body(src_hbm, idx_hbm, dst_hbm, vidx, dstv, rows, acc, sem,
                    i0, lo, hi, garbage, base=None, limit=None):
    i0 = pl.multiple_of(i0, 8)
    pltpu.sync_copy(idx_hbm.at[pl.ds(i0, CH)], vidx)
    pltpu.sync_copy(dst_hbm.at[pl.ds(i0, CH)], dstv)
    _mask_chunk(dstv, i0, lo, hi, garbage, base, limit)
    pltpu.async_copy(src_hbm.at[vidx], rows, sem).wait()
    pltpu.sync_copy(rows, acc.at[dstv], add=True)


def _edge_segsum_kernel():
    mesh = plsc.VectorSubcoreMesh(core_axis_name="c", subcore_axis_name="s")
    rows_per_tile = W // NS
    GD = 4
    SB = 8

    @functools.partial(
        pl.kernel,
        out_type=jax.ShapeDtypeStruct((NWIN * W, D), jnp.float32),
        mesh=mesh,
        scratch_types=[
            pltpu.VMEM((SB * CH,), jnp.int32),
            pltpu.VMEM((SB * CH,), jnp.int32),
            pltpu.VMEM((GD, CH), jnp.int32),
            pltpu.VMEM((CH, D), jnp.float32),
            pltpu.VMEM((CH, D), jnp.float32),
            pltpu.VMEM((CH, D), jnp.float32),
            pltpu.VMEM((CH, D), jnp.float32),
            pltpu.VMEM((rows_per_tile, D), jnp.float32),
            pltpu.VMEM((96,), jnp.int32),
            pltpu.VMEM_SHARED((W + 8, D), jnp.float32),
            pltpu.SemaphoreType.DMA,
            pltpu.SemaphoreType.DMA,
        ],
    )
    def k(x_hbm, vert_hbm, hl_hbm, ws_hbm, xe_hbm,
          sidx, sdst, dstv2, r0, r1, r2, r3, zbuf, wsv, acc, gsem, ssem):
        rows_b = (r0, r1, r2, r3)
        c = lax.axis_index("c")
        s = lax.axis_index("s")
        pltpu.sync_copy(ws_hbm, wsv)
        _zero_vmem_rows(zbuf, rows_per_tile)
        my_rows = s * rows_per_tile

        def gather_desc(kl, b):
            return pltpu.make_async_copy(
                x_hbm.at[sidx.at[pl.ds(kl * CH, CH)]], rows_b[b], gsem)

        def scatter_desc(b):
            return pltpu.make_async_copy(rows_b[b], acc.at[dstv2.at[b]], ssem)

        def window_body(wi, _):
            w = wi * NC + c
            lo_all = _extract(wsv, w)
            hi_all = _extract(wsv, w + 1)
            pltpu.sync_copy(zbuf, acc.at[pl.ds(my_rows, rows_per_tile)])
            plsc.subcore_barrier()
            cnt = hi_all - lo_all
            lo = lo_all + ((s * cnt) >> 4)
            hi = lo_all + (((s + 1) * cnt) >> 4)
            lo8 = lo & (-8)
            nch = (hi - lo8 + CH - 1) >> 7
            nst = (nch + SB - 1) >> 3

            def mask_into(kk, kl, b, lo=lo, hi=hi, lo8=lo8):
                i16 = lax.iota(jnp.int32, 16)
                for q in range(CH // L):
                    dv = sdst[pl.ds(kl * CH + q * L, L)]
                    pos = lo8 + kk * CH + q * L + i16
                    ok = (pos >= lo) & (pos < hi)
                    dstv2[b, pl.ds(q * L, L)] = jnp.where(ok, dv, W)

            def stage_blk(st, _, lo8=lo8, nch=nch):
                i0 = pl.multiple_of(lo8 + st * (SB * CH), 8)
                pltpu.sync_copy(vert_hbm.at[pl.ds(i0, SB * CH)], sidx)
                pltpu.sync_copy(hl_hbm.at[pl.ds(i0, SB * CH)], sdst)
                for gsub in range(SB // GD):
                    g = st * (SB // GD) + gsub
                    for b in range(GD):
                        kk = g * GD + b
                        @pl.when((g >= 1) & ((g - 1) * GD + b < nch))
                        def _(b=b):
                            scatter_desc(b).wait()
                        @pl.when(kk < nch)
                        def _(kk=kk, kl=gsub * GD + b, b=b):
                            mask_into(kk, kl, b)
                            gather_desc(kl, b).start()
                    for b in range(GD):
                        kk = g * GD + b
                        @pl.when(kk < nch)
                        def _(kl=gsub * GD + b, b=b):
                            gather_desc(kl, b).wait()
                            scatter_desc(b).start(add=True)
                return 0

            lax.fori_loop(0, nst, stage_blk, 0)
            gl = nst * (SB // GD) - 1
            for b in range(GD):
                @pl.when((gl >= 0) & (gl * GD + b < nch))
                def _(b=b):
                    scatter_desc(b).wait()
            plsc.subcore_barrier()
            pltpu.sync_copy(acc.at[pl.ds(my_rows, rows_per_tile)],
                            xe_hbm.at[pl.ds(w * W + my_rows, rows_per_tile)])
            return 0

        lax.fori_loop(0, NWIN // NC, window_body, 0)

    return k


def _node_segsum_kernel():
    mesh = plsc.VectorSubcoreMesh(core_axis_name="c", subcore_axis_name="s")
    per_tile = I // NS
    z_rows = NACC // NS
    GD = 4
    NCH = 160
    HCH = NCH // 2
    IDXN = HCH * CH

    @functools.partial(
        pl.kernel,
        out_type=jax.ShapeDtypeStruct((NC, NACC, D), jnp.float32),
        mesh=mesh,
        scratch_types=[
            pltpu.VMEM((IDXN,), jnp.int32),
            pltpu.VMEM((IDXN,), jnp.int32),
            pltpu.VMEM((GD, CH), jnp.int32),
            pltpu.VMEM((CH, D), jnp.float32),
            pltpu.VMEM((CH, D), jnp.float32),
            pltpu.VMEM((CH, D), jnp.float32),
            pltpu.VMEM((CH, D), jnp.float32),
            pltpu.VMEM_SHARED((NACC, D), jnp.float32),
            pltpu.SemaphoreType.DMA,
            pltpu.SemaphoreType.DMA,
        ],
    )
    def k(ue_hbm, he_hbm, vert_hbm, xv_hbm, gidx, didx, dstv2, r0, r1, r2, r3,
          acc, gsem, ssem):
        rows_b = (r0, r1, r2, r3)
        c = lax.axis_index("c")
        s = lax.axis_index("s")
        lo = s * per_tile
        hi = lo + per_tile
        base = c * HALF

        for b in range(GD):
            _zero_vmem_rows(rows_b[b], CH if b < 2 else z_rows - 2 * CH)
        pltpu.sync_copy(r0, acc.at[pl.ds(s * z_rows, CH)])
        pltpu.sync_copy(r1, acc.at[pl.ds(s * z_rows + CH, CH)])
        pltpu.sync_copy(r2.at[pl.ds(0, z_rows - 2 * CH)],
                        acc.at[pl.ds(s * z_rows + 2 * CH, z_rows - 2 * CH)])
        plsc.subcore_barrier()
        pltpu.sync_copy(he_hbm.at[pl.ds(lo, IDXN)], gidx)
        pltpu.sync_copy(vert_hbm.at[pl.ds(lo, IDXN)], didx)

        def gather_desc(kl, b):
            off = pl.multiple_of(kl * CH, CH)
            return pltpu.make_async_copy(
                ue_hbm.at[gidx.at[pl.ds(off, CH)]], rows_b[b], gsem)

        def scatter_desc(b):
            return pltpu.make_async_copy(rows_b[b], acc.at[dstv2.at[b]], ssem)

        def mask_into(kk, kl, b):
            i16 = lax.iota(jnp.int32, 16)
            koff = pl.multiple_of(kl * CH, CH)
            for q in range(CH // L):
                dv = didx[pl.ds(koff + q * L, L)] - base
                pos = lo + kk * CH + q * L + i16
                ok = (pos < hi) & (dv >= 0) & (dv < HALF)
                dstv2[b, pl.ds(q * L, L)] = jnp.where(ok, dv, HALF)

        half_groups = HCH // GD

        def group(gi, _):
            @pl.when(gi == half_groups)
            def _():
                pltpu.sync_copy(he_hbm.at[pl.ds(lo + IDXN, IDXN)], gidx)
                pltpu.sync_copy(vert_hbm.at[pl.ds(lo + IDXN, IDXN)], didx)
            k0 = gi * GD
            kl0 = jnp.where(gi >= half_groups, k0 - HCH, k0)
            for b in range(GD):
                @pl.when(gi > 0)
                def _():
                    scatter_desc(b).wait()
                mask_into(k0 + b, kl0 + b, b)
                gather_desc(kl0 + b, b).start()
            for b in range(GD):
                gather_desc(kl0 + b, b).wait()
                scatter_desc(b).start(add=True)
            return 0

        lax.fori_loop(0, NCH // GD, group, 0)
        for b in range(GD):
            scatter_desc(b).wait()
        plsc.subcore_barrier()
        pltpu.sync_copy(acc.at[pl.ds(s * z_rows, z_rows)],
                        xv_hbm.at[c, pl.ds(s * z_rows, z_rows)])

    return k


BE = 640
BN = 1000


def _edge_mlp_body(xe, ef, w0a, w0b, b0, w1, b1, w2, b2, g, bt, upd, eo):
    h = jnp.dot(xe[...], w0a[...], preferred_element_type=jnp.float32)
    h = h + jnp.dot(ef[...], w0b[...], preferred_element_type=jnp.float32)
    h = jnp.maximum(h + b0[...], 0.0)
    h = jnp.maximum(jnp.dot(h, w1[...], preferred_element_type=jnp.float32) + b1[...], 0.0)
    h = jnp.dot(h, w2[...], preferred_element_type=jnp.float32) + b2[...]
    m = jnp.mean(h, axis=-1, keepdims=True)
    v = jnp.mean((h - m) ** 2, axis=-1, keepdims=True)
    u = (h - m) * lax.rsqrt(v + 1e-5) * g[...] + bt[...]
    upd[...] = u
    eo[...] = u + ef[...]


def _node_mlp_body(xv, x, w0a, w0b, b0, w1, b1, w2, b2, g, bt, xo):
    h = jnp.dot(xv[0], w0a[...], preferred_element_type=jnp.float32)
    h = h + jnp.dot(x[...], w0b[...], preferred_element_type=jnp.float32)
    h = jnp.maximum(h + b0[...], 0.0)
    h = jnp.maximum(jnp.dot(h, w1[...], preferred_element_type=jnp.float32) + b1[...], 0.0)
    h = jnp.dot(h, w2[...], preferred_element_type=jnp.float32) + b2[...]
    m = jnp.mean(h, axis=-1, keepdims=True)
    v = jnp.mean((h - m) ** 2, axis=-1, keepdims=True)
    u = (h - m) * lax.rsqrt(v + 1e-5) * g[...] + bt[...]
    xo[...] = u + x[...]


def _row_spec(rows):
    return pl.BlockSpec((rows, D), lambda i: (i, 0))


def _full_spec(shape):
    return pl.BlockSpec(shape, lambda i: tuple(0 for _ in shape))


def _edge_mlp(xe, ef, w0a, w0b, b0, w1, b1, w2, b2, g, bt):
    specs = ([_row_spec(BE), _row_spec(BE)]
             + [_full_spec(w.shape) for w in (w0a, w0b, b0, w1, b1, w2, b2, g, bt)])
    return pl.pallas_call(
        _edge_mlp_body,
        grid=(E // BE,),
        in_specs=specs,
        out_specs=(_row_spec(BE), _row_spec(BE)),
        out_shape=(jax.ShapeDtypeStruct((E, D), jnp.float32),
                   jax.ShapeDtypeStruct((E, D), jnp.float32)),
        compiler_params=pltpu.CompilerParams(dimension_semantics=("parallel",)),
    )(xe, ef, w0a, w0b, b0, w1, b1, w2, b2, g, bt)


def _node_mlp(xv, x, w0a, w0b, b0, w1, b1, w2, b2, g, bt):
    nb = HALF // BN
    xv_spec = pl.BlockSpec((1, BN, D), lambda i: (i // nb, i % nb, 0))
    specs = ([xv_spec, _row_spec(BN)]
             + [_full_spec(w.shape) for w in (w0a, w0b, b0, w1, b1, w2, b2, g, bt)])
    return pl.pallas_call(
        _node_mlp_body,
        grid=(N // BN,),
        in_specs=specs,
        out_specs=_row_spec(BN),
        out_shape=jax.ShapeDtypeStruct((N, D), jnp.float32),
        compiler_params=pltpu.CompilerParams(dimension_semantics=("parallel",)),
    )(xv, x, w0a, w0b, b0, w1, b1, w2, b2, g, bt)


def kernel(x, vertices, hyperedges, edge_features,
           eW0, eb0, eW1, eb1, eW2, eb2, eg, ebt,
           nW0, nb0, nW1, nb1, nW2, nb2, ng, nbt):
    S = eW0.shape[0]
    ipad = jnp.zeros((PAD,), jnp.int32)
    vert_p = jnp.concatenate([vertices, ipad])
    he_p = jnp.concatenate([hyperedges, ipad])
    hl_p = jnp.concatenate([hyperedges % W, ipad])
    bnd = jnp.searchsorted(
        hyperedges, jnp.arange(NWIN + 1, dtype=jnp.int32) * W, side='left'
    ).astype(jnp.int32)
    ws = jnp.full((96,), I, jnp.int32).at[:NWIN + 1].set(bnd)

    edge_seg = _edge_segsum_kernel()
    node_seg = _node_segsum_kernel()

    for s in range(S):
        xe = edge_seg(x, vert_p, hl_p, ws)
        upd_e, e_out = _edge_mlp(
            xe, edge_features,
            eW0[s][:D], eW0[s][D:], eb0[s][None, :],
            eW1[s], eb1[s][None, :], eW2[s], eb2[s][None, :],
            eg[s][None, :], ebt[s][None, :])
        xv = node_seg(upd_e, he_p, vert_p)
        x = _node_mlp(
            xv, x,
            nW0[s][:D], nW0[s][D:], nb0[s][None, :],
            nW1[s], nb1[s][None, :], nW2[s], nb2[s][None, :],
            ng[s][None, :], nbt[s][None, :])
        edge_features = e_out

    return (x, edge_features)


# speedup vs baseline: 3.6844x; 1.1165x over previous
import functools

import jax
import jax.numpy as jnp
from jax import lax
from jax.experimental import pallas as pl
from jax.experimental.pallas import tpu as pltpu
from jax.experimental.pallas import tpu_sc as plsc

N = 10000
I = 320000
E = 160000
D = 128
NC = 2
NS = 16
W = 4096
NWIN = 40
HALF = N // NC
NACC = 5120
CH = 128
PAD = 2048
L = 16


def _zero_vmem_rows(buf, nrows):
    z = jnp.zeros((L,), jnp.float32)

    def row(r, _):
        for q in range(D // L):
            buf[r, pl.ds(q * L, L)] = z
        return 0

    lax.fori_loop(0, nrows, row, 0)


def _extract(wsv, j):
    return wsv[pl.ds(j, L)][0]


def _mask_chunk(dstv, i0, lo, hi, garbage, base=None, limit=None):
    i16 = lax.iota(jnp.int32, 16)
    for q in range(CH // L):
        dv = dstv[pl.ds(q * L, L)]
        pos = i0 + q * L + i16
        ok = (pos >= lo) & (pos < hi)
        if base is not None:
            dv = dv - base
            ok = ok & (dv >= 0) & (dv < limit)
        dstv[pl.ds(q * L, L)] = jnp.where(ok, dv, garbage)


def _seg_chunk_body(src_hbm, idx_hbm, dst_hbm, vidx, dstv, rows, acc, sem,
                    i0, lo, hi, garbage, base=None, limit=None):
    i0 = pl.multiple_of(i0, 8)
    pltpu.sync_copy(idx_hbm.at[pl.ds(i0, CH)], vidx)
    pltpu.sync_copy(dst_hbm.at[pl.ds(i0, CH)], dstv)
    _mask_chunk(dstv, i0, lo, hi, garbage, base, limit)
    pltpu.async_copy(src_hbm.at[vidx], rows, sem).wait()
    pltpu.sync_copy(rows, acc.at[dstv], add=True)


def _edge_segsum_kernel():
    mesh = plsc.VectorSubcoreMesh(core_axis_name="c", subcore_axis_name="s")
    rows_per_tile = W // NS
    GD = 4
    SB = 8
    ZR = 128

    @functools.partial(
        pl.kernel,
        out_type=jax.ShapeDtypeStruct((NWIN * W, D), jnp.float32),
        mesh=mesh,
        scratch_types=[
            pltpu.VMEM((SB * CH,), jnp.int32),
            pltpu.VMEM((SB * CH,), jnp.int32),
            pltpu.VMEM((GD, CH), jnp.int32),
            pltpu.VMEM((CH, D), jnp.float32),
            pltpu.VMEM((CH, D), jnp.float32),
            pltpu.VMEM((CH, D), jnp.float32),
            pltpu.VMEM((CH, D), jnp.float32),
            pltpu.VMEM((ZR, D), jnp.float32),
            pltpu.VMEM((96,), jnp.int32),
            pltpu.VMEM_SHARED((W + 8, D), jnp.float32),
            pltpu.SemaphoreType.DMA,
            pltpu.SemaphoreType.DMA,
        ],
    )
    def k(x_hbm, vert_hbm, hl_hbm, ws_hbm, xe_hbm,
          sidx, sdst, dstv2, r0, r1, r2, r3, zbuf, wsv, acc, gsem, ssem):
        rows_b = (r0, r1, r2, r3)
        c = lax.axis_index("c")
        s = lax.axis_index("s")
        pltpu.sync_copy(ws_hbm, wsv)
        _zero_vmem_rows(zbuf, ZR)
        my_rows = s * rows_per_tile

        def gather_desc(kl, b):
            return pltpu.make_async_copy(
                x_hbm.at[sidx.at[pl.ds(kl * CH, CH)]], rows_b[b], gsem)

        def scatter_desc(b):
            return pltpu.make_async_copy(rows_b[b], acc.at[dstv2.at[b]], ssem)

        def window_body(wi, _):
            w = wi * NC + c
            lo_all = _extract(wsv, w)
            hi_all = _extract(wsv, w + 1)
            for z in range(rows_per_tile // ZR):
                pltpu.sync_copy(zbuf, acc.at[pl.ds(my_rows + z * ZR, ZR)])
            plsc.subcore_barrier()
            cnt = hi_all - lo_all
            lo = lo_all + ((s * cnt) >> 4)
            hi = lo_all + (((s + 1) * cnt) >> 4)
            lo8 = lo & (-8)
            nch = (hi - lo8 + CH - 1) >> 7
            nst = (nch + SB - 1) >> 3

            def mask_into(kk, kl, b, lo=lo, hi=hi, lo8=lo8):
                i16 = lax.iota(jnp.int32, 16)
                for q in range(CH // L):
                    dv = sdst[pl.ds(kl * CH + q * L, L)]
                    pos = lo8 + kk * CH + q * L + i16
                    ok = (pos >= lo) & (pos < hi)
                    dstv2[b, pl.ds(q * L, L)] = jnp.where(ok, dv, W)

            def stage_blk(st, _, lo8=lo8, nch=nch):
                i0 = pl.multiple_of(lo8 + st * (SB * CH), 8)
                pltpu.sync_copy(vert_hbm.at[pl.ds(i0, SB * CH)], sidx)
                pltpu.sync_copy(hl_hbm.at[pl.ds(i0, SB * CH)], sdst)
                for gsub in range(SB // GD):
                    g = st * (SB // GD) + gsub
                    for b in range(GD):
                        kk = g * GD + b
                        @pl.when((g >= 1) & ((g - 1) * GD + b < nch))
                        def _(b=b):
                            scatter_desc(b).wait()
                        @pl.when(kk < nch)
                        def _(kk=kk, kl=gsub * GD + b, b=b):
                            mask_into(kk, kl, b)
                            gather_desc(kl, b).start()
                    for b in range(GD):
                        kk = g * GD + b
                        @pl.when(kk < nch)
                        def _(kl=gsub * GD + b, b=b):
                            gather_desc(kl, b).wait()
                            scatter_desc(b).start(add=True)
                return 0

            lax.fori_loop(0, nst, stage_blk, 0)
            gl = nst * (SB // GD) - 1
            for b in range(GD):
                @pl.when((gl >= 0) & (gl * GD + b < nch))
                def _(b=b):
                    scatter_desc(b).wait()
            plsc.subcore_barrier()
            pltpu.sync_copy(acc.at[pl.ds(my_rows, rows_per_tile)],
                            xe_hbm.at[pl.ds(w * W + my_rows, rows_per_tile)])
            return 0

        lax.fori_loop(0, NWIN // NC, window_body, 0)

    return k


def _node_segsum_kernel():
    mesh = plsc.VectorSubcoreMesh(core_axis_name="c", subcore_axis_name="s")
    per_tile = I // NS
    z_rows = NACC // NS
    GD = 4
    NCH = 160
    HCH = NCH // 2
    IDXN = HCH * CH

    @functools.partial(
        pl.kernel,
        out_type=jax.ShapeDtypeStruct((NC, NACC, D), jnp.float32),
        mesh=mesh,
        scratch_types=[
            pltpu.VMEM((IDXN,), jnp.int32),
            pltpu.VMEM((IDXN,), jnp.int32),
            pltpu.VMEM((GD, CH), jnp.int32),
            pltpu.VMEM((CH, D), jnp.float32),
            pltpu.VMEM((CH, D), jnp.float32),
            pltpu.VMEM((CH, D), jnp.float32),
            pltpu.VMEM((CH, D), jnp.float32),
            pltpu.VMEM_SHARED((NACC, D), jnp.float32),
            pltpu.SemaphoreType.DMA,
            pltpu.SemaphoreType.DMA,
        ],
    )
    def k(ue_hbm, he_hbm, vert_hbm, xv_hbm, gidx, didx, dstv2, r0, r1, r2, r3,
          acc, gsem, ssem):
        rows_b = (r0, r1, r2, r3)
        c = lax.axis_index("c")
        s = lax.axis_index("s")
        lo = s * per_tile
        hi = lo + per_tile
        base = c * HALF

        for b in range(GD):
            _zero_vmem_rows(rows_b[b], CH if b < 2 else z_rows - 2 * CH)
        pltpu.sync_copy(r0, acc.at[pl.ds(s * z_rows, CH)])
        pltpu.sync_copy(r1, acc.at[pl.ds(s * z_rows + CH, CH)])
        pltpu.sync_copy(r2.at[pl.ds(0, z_rows - 2 * CH)],
                        acc.at[pl.ds(s * z_rows + 2 * CH, z_rows - 2 * CH)])
        plsc.subcore_barrier()
        pltpu.sync_copy(he_hbm.at[pl.ds(lo, IDXN)], gidx)
        pltpu.sync_copy(vert_hbm.at[pl.ds(lo, IDXN)], didx)

        def gather_desc(kl, b):
            off = pl.multiple_of(kl * CH, CH)
            return pltpu.make_async_copy(
                ue_hbm.at[gidx.at[pl.ds(off, CH)]], rows_b[b], gsem)

        def scatter_desc(b):
            return pltpu.make_async_copy(rows_b[b], acc.at[dstv2.at[b]], ssem)

        def mask_into(kk, kl, b):
            i16 = lax.iota(jnp.int32, 16)
            koff = pl.multiple_of(kl * CH, CH)
            for q in range(CH // L):
                dv = didx[pl.ds(koff + q * L, L)] - base
                pos = lo + kk * CH + q * L + i16
                ok = (pos < hi) & (dv >= 0) & (dv < HALF)
                dstv2[b, pl.ds(q * L, L)] = jnp.where(ok, dv, HALF)

        half_groups = HCH // GD

        def group(gi, _):
            @pl.when(gi == half_groups)
            def _():
                pltpu.sync_copy(he_hbm.at[pl.ds(lo + IDXN, IDXN)], gidx)
                pltpu.sync_copy(vert_hbm.at[pl.ds(lo + IDXN, IDXN)], didx)
            k0 = gi * GD
            kl0 = jnp.where(gi >= half_groups, k0 - HCH, k0)
            for b in range(GD):
                @pl.when(gi > 0)
                def _():
                    scatter_desc(b).wait()
                mask_into(k0 + b, kl0 + b, b)
                gather_desc(kl0 + b, b).start()
            for b in range(GD):
                gather_desc(kl0 + b, b).wait()
                scatter_desc(b).start(add=True)
            return 0

        lax.fori_loop(0, NCH // GD, group, 0)
        for b in range(GD):
            scatter_desc(b).wait()
        plsc.subcore_barrier()
        pltpu.sync_copy(acc.at[pl.ds(s * z_rows, z_rows)],
                        xv_hbm.at[c, pl.ds(s * z_rows, z_rows)])

    return k


BE = 640
BN = 1000


def _edge_mlp_body(xe, ef, w0a, w0b, b0, w1, b1, w2, b2, g, bt, upd, eo):
    h = jnp.dot(xe[...], w0a[...], preferred_element_type=jnp.float32)
    h = h + jnp.dot(ef[...], w0b[...], preferred_element_type=jnp.float32)
    h = jnp.maximum(h + b0[...], 0.0)
    h = jnp.maximum(jnp.dot(h, w1[...], preferred_element_type=jnp.float32) + b1[...], 0.0)
    h = jnp.dot(h, w2[...], preferred_element_type=jnp.float32) + b2[...]
    m = jnp.mean(h, axis=-1, keepdims=True)
    v = jnp.mean((h - m) ** 2, axis=-1, keepdims=True)
    u = (h - m) * lax.rsqrt(v + 1e-5) * g[...] + bt[...]
    upd[...] = u
    eo[...] = u + ef[...]


def _node_mlp_body(xv, x, w0a, w0b, b0, w1, b1, w2, b2, g, bt, xo):
    h = jnp.dot(xv[0], w0a[...], preferred_element_type=jnp.float32)
    h = h + jnp.dot(x[...], w0b[...], preferred_element_type=jnp.float32)
    h = jnp.maximum(h + b0[...], 0.0)
    h = jnp.maximum(jnp.dot(h, w1[...], preferred_element_type=jnp.float32) + b1[...], 0.0)
    h = jnp.dot(h, w2[...], preferred_element_type=jnp.float32) + b2[...]
    m = jnp.mean(h, axis=-1, keepdims=True)
    v = jnp.mean((h - m) ** 2, axis=-1, keepdims=True)
    u = (h - m) * lax.rsqrt(v + 1e-5) * g[...] + bt[...]
    xo[...] = u + x[...]


def _row_spec(rows):
    return pl.BlockSpec((rows, D), lambda i: (i, 0))


def _full_spec(shape):
    return pl.BlockSpec(shape, lambda i: tuple(0 for _ in shape))


def _edge_mlp(xe, ef, w0a, w0b, b0, w1, b1, w2, b2, g, bt):
    specs = ([_row_spec(BE), _row_spec(BE)]
             + [_full_spec(w.shape) for w in (w0a, w0b, b0, w1, b1, w2, b2, g, bt)])
    return pl.pallas_call(
        _edge_mlp_body,
        grid=(E // BE,),
        in_specs=specs,
        out_specs=(_row_spec(BE), _row_spec(BE)),
        out_shape=(jax.ShapeDtypeStruct((E, D), jnp.float32),
                   jax.ShapeDtypeStruct((E, D), jnp.float32)),
        compiler_params=pltpu.CompilerParams(dimension_semantics=("parallel",)),
    )(xe, ef, w0a, w0b, b0, w1, b1, w2, b2, g, bt)


def _node_mlp(xv, x, w0a, w0b, b0, w1, b1, w2, b2, g, bt):
    nb = HALF // BN
    xv_spec = pl.BlockSpec((1, BN, D), lambda i: (i // nb, i % nb, 0))
    specs = ([xv_spec, _row_spec(BN)]
             + [_full_spec(w.shape) for w in (w0a, w0b, b0, w1, b1, w2, b2, g, bt)])
    return pl.pallas_call(
        _node_mlp_body,
        grid=(N // BN,),
        in_specs=specs,
        out_specs=_row_spec(BN),
        out_shape=jax.ShapeDtypeStruct((N, D), jnp.float32),
        compiler_params=pltpu.CompilerParams(dimension_semantics=("parallel",)),
    )(xv, x, w0a, w0b, b0, w1, b1, w2, b2, g, bt)


def kernel(x, vertices, hyperedges, edge_features,
           eW0, eb0, eW1, eb1, eW2, eb2, eg, ebt,
           nW0, nb0, nW1, nb1, nW2, nb2, ng, nbt):
    S = eW0.shape[0]
    ipad = jnp.zeros((PAD,), jnp.int32)
    vert_p = jnp.concatenate([vertices, ipad])
    he_p = jnp.concatenate([hyperedges, ipad])
    hl_p = jnp.concatenate([hyperedges % W, ipad])
    bnd = jnp.searchsorted(
        hyperedges, jnp.arange(NWIN + 1, dtype=jnp.int32) * W, side='left'
    ).astype(jnp.int32)
    ws = jnp.full((96,), I, jnp.int32).at[:NWIN + 1].set(bnd)

    edge_seg = _edge_segsum_kernel()
    node_seg = _node_segsum_kernel()

    for s in range(S):
        xe = edge_seg(x, vert_p, hl_p, ws)
        upd_e, e_out = _edge_mlp(
            xe, edge_features,
            eW0[s][:D], eW0[s][D:], eb0[s][None, :],
            eW1[s], eb1[s][None, :], eW2[s], eb2[s][None, :],
            eg[s][None, :], ebt[s][None, :])
        xv = node_seg(upd_e, he_p, vert_p)
        x = _node_mlp(
            xv, x,
            nW0[s][:D], nW0[s][D:], nb0[s][None, :],
            nW1[s], nb1[s][None, :], nW2[s], nb2[s][None, :],
            ng[s][None, :], nbt[s][None, :])
        edge_features = e_out

    return (x, edge_features)


# speedup vs baseline: 4.5299x; 1.2295x over previous
import functools

import jax
import jax.numpy as jnp
from jax import lax
from jax.experimental import pallas as pl
from jax.experimental.pallas import tpu as pltpu
from jax.experimental.pallas import tpu_sc as plsc

N = 10000
I = 320000
E = 160000
D = 128
NC = 2
NS = 16
W = 4096
NWIN = 40
HALF = N // NC
NACC = 10240
CH = 128
PAD = 2048
L = 16


def _zero_vmem_rows(buf, nrows):
    z = jnp.zeros((L,), jnp.float32)

    def row(r, _):
        for q in range(D // L):
            buf[r, pl.ds(q * L, L)] = z
        return 0

    lax.fori_loop(0, nrows, row, 0)


def _extract(wsv, j):
    return wsv[pl.ds(j, L)][0]


def _mask_chunk(dstv, i0, lo, hi, garbage, base=None, limit=None):
    i16 = lax.iota(jnp.int32, 16)
    for q in range(CH // L):
        dv = dstv[pl.ds(q * L, L)]
        pos = i0 + q * L + i16
        ok = (pos >= lo) & (pos < hi)
        if base is not None:
            dv = dv - base
            ok = ok & (dv >= 0) & (dv < limit)
        dstv[pl.ds(q * L, L)] = jnp.where(ok, dv, garbage)


def _seg_chunk_body(src_hbm, idx_hbm, dst_hbm, vidx, dstv, rows, acc, sem,
                    i0, lo, hi, garbage, base=None, limit=None):
    i0 = pl.multiple_of(i0, 8)
    pltpu.sync_copy(idx_hbm.at[pl.ds(i0, CH)], vidx)
    pltpu.sync_copy(dst_hbm.at[pl.ds(i0, CH)], dstv)
    _mask_chunk(dstv, i0, lo, hi, garbage, base, limit)
    pltpu.async_copy(src_hbm.at[vidx], rows, sem).wait()
    pltpu.sync_copy(rows, acc.at[dstv], add=True)


def _edge_segsum_kernel():
    mesh = plsc.VectorSubcoreMesh(core_axis_name="c", subcore_axis_name="s")
    rows_per_tile = W // NS
    GD = 4
    SB = 8
    ZR = 128

    @functools.partial(
        pl.kernel,
        out_type=jax.ShapeDtypeStruct((NWIN * W, D), jnp.float32),
        mesh=mesh,
        scratch_types=[
            pltpu.VMEM((SB * CH,), jnp.int32),
            pltpu.VMEM((SB * CH,), jnp.int32),
            pltpu.VMEM((GD, CH), jnp.int32),
            pltpu.VMEM((CH, D), jnp.float32),
            pltpu.VMEM((CH, D), jnp.float32),
            pltpu.VMEM((CH, D), jnp.float32),
            pltpu.VMEM((CH, D), jnp.float32),
            pltpu.VMEM((ZR, D), jnp.float32),
            pltpu.VMEM((96,), jnp.int32),
            pltpu.VMEM_SHARED((W + 8, D), jnp.float32),
            pltpu.SemaphoreType.DMA,
            pltpu.SemaphoreType.DMA,
        ],
    )
    def k(x_hbm, vert_hbm, hl_hbm, ws_hbm, xe_hbm,
          sidx, sdst, dstv2, r0, r1, r2, r3, zbuf, wsv, acc, gsem, ssem):
        rows_b = (r0, r1, r2, r3)
        c = lax.axis_index("c")
        s = lax.axis_index("s")
        pltpu.sync_copy(ws_hbm, wsv)
        _zero_vmem_rows(zbuf, ZR)
        my_rows = s * rows_per_tile

        def gather_desc(kl, b):
            return pltpu.make_async_copy(
                x_hbm.at[sidx.at[pl.ds(kl * CH, CH)]], rows_b[b], gsem)

        def scatter_desc(b):
            return pltpu.make_async_copy(rows_b[b], acc.at[dstv2.at[b]], ssem)

        def window_body(wi, _):
            w = wi * NC + c
            lo_all = _extract(wsv, w)
            hi_all = _extract(wsv, w + 1)
            for z in range(rows_per_tile // ZR):
                pltpu.sync_copy(zbuf, acc.at[pl.ds(my_rows + z * ZR, ZR)])
            plsc.subcore_barrier()
            cnt = hi_all - lo_all
            lo = lo_all + ((s * cnt) >> 4)
            hi = lo_all + (((s + 1) * cnt) >> 4)
            lo8 = lo & (-8)
            nch = (hi - lo8 + CH - 1) >> 7
            nst = (nch + SB - 1) >> 3

            def mask_into(kk, kl, b, lo=lo, hi=hi, lo8=lo8):
                i16 = lax.iota(jnp.int32, 16)
                for q in range(CH // L):
                    dv = sdst[pl.ds(kl * CH + q * L, L)]
                    pos = lo8 + kk * CH + q * L + i16
                    ok = (pos >= lo) & (pos < hi)
                    dstv2[b, pl.ds(q * L, L)] = jnp.where(ok, dv, W)

            def stage_blk(st, _, lo8=lo8, nch=nch):
                i0 = pl.multiple_of(lo8 + st * (SB * CH), 8)
                pltpu.sync_copy(vert_hbm.at[pl.ds(i0, SB * CH)], sidx)
                pltpu.sync_copy(hl_hbm.at[pl.ds(i0, SB * CH)], sdst)
                for gsub in range(SB // GD):
                    g = st * (SB // GD) + gsub
                    for b in range(GD):
                        kk = g * GD + b
                        @pl.when((g >= 1) & ((g - 1) * GD + b < nch))
                        def _(b=b):
                            scatter_desc(b).wait()
                        @pl.when(kk < nch)
                        def _(kk=kk, kl=gsub * GD + b, b=b):
                            mask_into(kk, kl, b)
                            gather_desc(kl, b).start()
                    for b in range(GD):
                        kk = g * GD + b
                        @pl.when(kk < nch)
                        def _(kl=gsub * GD + b, b=b):
                            gather_desc(kl, b).wait()
                            scatter_desc(b).start(add=True)
                return 0

            lax.fori_loop(0, nst, stage_blk, 0)
            gl = nst * (SB // GD) - 1
            for b in range(GD):
                @pl.when((gl >= 0) & (gl * GD + b < nch))
                def _(b=b):
                    scatter_desc(b).wait()
            plsc.subcore_barrier()
            pltpu.sync_copy(acc.at[pl.ds(my_rows, rows_per_tile)],
                            xe_hbm.at[pl.ds(w * W + my_rows, rows_per_tile)])
            return 0

        lax.fori_loop(0, NWIN // NC, window_body, 0)

    return k


def _node_segsum_kernel():
    mesh = plsc.VectorSubcoreMesh(core_axis_name="c", subcore_axis_name="s")
    per_tile = I // (NC * NS)
    z_rows = NACC // NS
    GD = 2
    NCH = 80
    HCH = NCH // 2
    IDXN = HCH * CH

    @functools.partial(
        pl.kernel,
        out_type=jax.ShapeDtypeStruct((NC, NACC, D), jnp.float32),
        mesh=mesh,
        scratch_types=[
            pltpu.VMEM((IDXN,), jnp.int32),
            pltpu.VMEM((IDXN,), jnp.int32),
            pltpu.VMEM((GD, CH), jnp.int32),
            pltpu.VMEM((CH, D), jnp.float32),
            pltpu.VMEM((CH, D), jnp.float32),
            pltpu.VMEM_SHARED((NACC, D), jnp.float32),
            pltpu.SemaphoreType.DMA,
            pltpu.SemaphoreType.DMA,
        ],
    )
    def k(ue_hbm, he_hbm, vert_hbm, xv_hbm, gidx, didx, dstv2, r0, r1,
          acc, gsem, ssem):
        rows_b = (r0, r1)
        c = lax.axis_index("c")
        s = lax.axis_index("s")
        lo = c * (I // NC) + s * per_tile
        hi = lo + per_tile

        _zero_vmem_rows(r0, CH)
        _zero_vmem_rows(r1, CH)
        for z in range(z_rows // CH):
            pltpu.sync_copy(r0 if z % 2 == 0 else r1,
                            acc.at[pl.ds(s * z_rows + z * CH, CH)])
        plsc.subcore_barrier()
        pltpu.sync_copy(he_hbm.at[pl.ds(lo, IDXN)], gidx)
        pltpu.sync_copy(vert_hbm.at[pl.ds(lo, IDXN)], didx)

        def gather_desc(kl, b):
            off = pl.multiple_of(kl * CH, CH)
            return pltpu.make_async_copy(
                ue_hbm.at[gidx.at[pl.ds(off, CH)]], rows_b[b], gsem)

        def scatter_desc(b):
            return pltpu.make_async_copy(rows_b[b], acc.at[dstv2.at[b]], ssem)

        def mask_into(kk, kl, b):
            i16 = lax.iota(jnp.int32, 16)
            koff = pl.multiple_of(kl * CH, CH)
            for q in range(CH // L):
                dv = didx[pl.ds(koff + q * L, L)]
                pos = lo + kk * CH + q * L + i16
                dstv2[b, pl.ds(q * L, L)] = jnp.where(pos < hi, dv, N)

        half_groups = HCH // GD

        def group(gi, _):
            @pl.when(gi == half_groups)
            def _():
                pltpu.sync_copy(he_hbm.at[pl.ds(lo + IDXN, IDXN)], gidx)
                pltpu.sync_copy(vert_hbm.at[pl.ds(lo + IDXN, IDXN)], didx)
            k0 = gi * GD
            kl0 = jnp.where(gi >= half_groups, k0 - HCH, k0)
            for b in range(GD):
                @pl.when(gi > 0)
                def _(b=b):
                    scatter_desc(b).wait()
                mask_into(k0 + b, kl0 + b, b)
                gather_desc(kl0 + b, b).start()
            for b in range(GD):
                gather_desc(kl0 + b, b).wait()
                scatter_desc(b).start(add=True)
            return 0

        lax.fori_loop(0, NCH // GD, group, 0)
        for b in range(GD):
            scatter_desc(b).wait()
        plsc.subcore_barrier()
        pltpu.sync_copy(acc.at[pl.ds(s * z_rows, z_rows)],
                        xv_hbm.at[c, pl.ds(s * z_rows, z_rows)])

    return k


BE = 640
BN = 1000


def _edge_mlp_body(xe, ef, w0a, w0b, b0, w1, b1, w2, b2, g, bt, upd, eo):
    h = jnp.dot(xe[...], w0a[...], preferred_element_type=jnp.float32)
    h = h + jnp.dot(ef[...], w0b[...], preferred_element_type=jnp.float32)
    h = jnp.maximum(h + b0[...], 0.0)
    h = jnp.maximum(jnp.dot(h, w1[...], preferred_element_type=jnp.float32) + b1[...], 0.0)
    h = jnp.dot(h, w2[...], preferred_element_type=jnp.float32) + b2[...]
    m = jnp.mean(h, axis=-1, keepdims=True)
    v = jnp.mean((h - m) ** 2, axis=-1, keepdims=True)
    u = (h - m) * lax.rsqrt(v + 1e-5) * g[...] + bt[...]
    upd[...] = u
    eo[...] = u + ef[...]


def _node_mlp_body(xv0, xv1, x, w0a, w0b, b0, w1, b1, w2, b2, g, bt, xo):
    h = jnp.dot(xv0[0] + xv1[0], w0a[...], preferred_element_type=jnp.float32)
    h = h + jnp.dot(x[...], w0b[...], preferred_element_type=jnp.float32)
    h = jnp.maximum(h + b0[...], 0.0)
    h = jnp.maximum(jnp.dot(h, w1[...], preferred_element_type=jnp.float32) + b1[...], 0.0)
    h = jnp.dot(h, w2[...], preferred_element_type=jnp.float32) + b2[...]
    m = jnp.mean(h, axis=-1, keepdims=True)
    v = jnp.mean((h - m) ** 2, axis=-1, keepdims=True)
    u = (h - m) * lax.rsqrt(v + 1e-5) * g[...] + bt[...]
    xo[...] = u + x[...]


def _row_spec(rows):
    return pl.BlockSpec((rows, D), lambda i: (i, 0))


def _full_spec(shape):
    return pl.BlockSpec(shape, lambda i: tuple(0 for _ in shape))


def _edge_mlp(xe, ef, w0a, w0b, b0, w1, b1, w2, b2, g, bt):
    specs = ([_row_spec(BE), _row_spec(BE)]
             + [_full_spec(w.shape) for w in (w0a, w0b, b0, w1, b1, w2, b2, g, bt)])
    return pl.pallas_call(
        _edge_mlp_body,
        grid=(E // BE,),
        in_specs=specs,
        out_specs=(_row_spec(BE), _row_spec(BE)),
        out_shape=(jax.ShapeDtypeStruct((E, D), jnp.float32),
                   jax.ShapeDtypeStruct((E, D), jnp.float32)),
        compiler_params=pltpu.CompilerParams(dimension_semantics=("parallel",)),
    )(xe, ef, w0a, w0b, b0, w1, b1, w2, b2, g, bt)


def _node_mlp(xv, x, w0a, w0b, b0, w1, b1, w2, b2, g, bt):
    xv0_spec = pl.BlockSpec((1, BN, D), lambda i: (0, i, 0))
    xv1_spec = pl.BlockSpec((1, BN, D), lambda i: (1, i, 0))
    specs = ([xv0_spec, xv1_spec, _row_spec(BN)]
             + [_full_spec(w.shape) for w in (w0a, w0b, b0, w1, b1, w2, b2, g, bt)])
    return pl.pallas_call(
        _node_mlp_body,
        grid=(N // BN,),
        in_specs=specs,
        out_specs=_row_spec(BN),
        out_shape=jax.ShapeDtypeStruct((N, D), jnp.float32),
        compiler_params=pltpu.CompilerParams(dimension_semantics=("parallel",)),
    )(xv, xv, x, w0a, w0b, b0, w1, b1, w2, b2, g, bt)


def kernel(x, vertices, hyperedges, edge_features,
           eW0, eb0, eW1, eb1, eW2, eb2, eg, ebt,
           nW0, nb0, nW1, nb1, nW2, nb2, ng, nbt):
    S = eW0.shape[0]
    ipad = jnp.zeros((PAD,), jnp.int32)
    vert_p = jnp.concatenate([vertices, ipad])
    he_p = jnp.concatenate([hyperedges, ipad])
    hl_p = jnp.concatenate([hyperedges % W, ipad])
    bnd = jnp.searchsorted(
        hyperedges, jnp.arange(NWIN + 1, dtype=jnp.int32) * W, side='left'
    ).astype(jnp.int32)
    ws = jnp.full((96,), I, jnp.int32).at[:NWIN + 1].set(bnd)

    edge_seg = _edge_segsum_kernel()
    node_seg = _node_segsum_kernel()

    for s in range(S):
        xe = edge_seg(x, vert_p, hl_p, ws)
        upd_e, e_out = _edge_mlp(
            xe, edge_features,
            eW0[s][:D], eW0[s][D:], eb0[s][None, :],
            eW1[s], eb1[s][None, :], eW2[s], eb2[s][None, :],
            eg[s][None, :], ebt[s][None, :])
        xv = node_seg(upd_e, he_p, vert_p)
        x = _node_mlp(
            xv, x,
            nW0[s][:D], nW0[s][D:], nb0[s][None, :],
            nW1[s], nb1[s][None, :], nW2[s], nb2[s][None, :],
            ng[s][None, :], nbt[s][None, :])
        edge_features = e_out

    return (x, edge_features)
